# Initial kernel scaffold; baseline (speedup 1.0000x reference)
import jax
import jax.numpy as jnp
from jax.experimental import pallas as pl


def kernel(x_user, x_movie, user_node_id, movie_node_id, edge_index, edge_label_index, W_user_lin, b_user_lin, W_movie_lin, b_movie_lin, user_emb, movie_emb, Wl1r, bl1r, Wr1r, Wl1v, bl1v, Wr1v, Wl2r, bl2r, Wr2r, Wl2v, bl2v, Wr2v):
    raise NotImplementedError("write your pallas kernel here")


# speedup vs baseline: 1.7886x; 1.7886x over previous
import functools

import jax
import jax.numpy as jnp
from jax import lax
from jax.experimental import pallas as pl
from jax.experimental.pallas import tpu as pltpu
from jax.experimental.pallas import tpu_sc as plsc

F32 = jnp.float32
H = 128
N_NODES = 10000
N_PAD = 10240
E = 320000
E_CHUNK = 128
N_TILES = 32
CT = 80
IG = 16
E_PAD = N_TILES * CT * E_CHUNK
CNT_LEN = 10240
EL = 100000
CTL = 25
EL_PAD = N_TILES * CTL * E_CHUNK

_mesh = plsc.VectorSubcoreMesh(core_axis_name="c", subcore_axis_name="s")


def _seg_body(gidx, sidx, table, zacc, out_s, gv, sv, rows, acc, sem):
    cid = lax.axis_index("c")
    sid = lax.axis_index("s")
    wid = cid * 16 + sid
    stripe = N_PAD // 16
    pltpu.sync_copy(zacc.at[pl.ds(sid * stripe, stripe)],
                    acc.at[pl.ds(sid * stripe, stripe)])
    plsc.subcore_barrier()

    def group(g, carry):
        pltpu.sync_copy(gidx.at[pl.ds(wid * CT + g * IG, IG)], gv)
        pltpu.sync_copy(sidx.at[pl.ds(wid * CT + g * IG, IG)], sv)

        def chunk(j, carry2):
            pltpu.async_copy(table.at[gv.at[j]], rows, sem).wait()
            pltpu.sync_copy(rows, acc.at[sv.at[j]], add=True)
            return carry2
        lax.fori_loop(0, IG, chunk, 0)
        return carry
    lax.fori_loop(0, CT // IG, group, 0)
    plsc.subcore_barrier()
    pltpu.sync_copy(acc.at[pl.ds(sid * stripe, stripe)],
                    out_s.at[pl.ds(cid * N_PAD + sid * stripe, stripe)])


_seg_call = functools.partial(
    pl.kernel, mesh=_mesh,
    out_type=[jax.ShapeDtypeStruct((2 * N_PAD, H), F32)],
    scratch_types=[
        pltpu.VMEM((IG, E_CHUNK), jnp.int32),
        pltpu.VMEM((IG, E_CHUNK), jnp.int32),
        pltpu.VMEM((E_CHUNK, H), F32),
        pltpu.VMEM_SHARED((N_PAD, H), F32),
        pltpu.SemaphoreType.DMA,
    ],
)(_seg_body)


def _cnt_body(sidx, zacc, ones_in, out_c, sv, onesv, acc):
    cid = lax.axis_index("c")
    sid = lax.axis_index("s")
    wid = cid * 16 + sid
    stripe = N_PAD // 16
    pltpu.sync_copy(zacc.at[pl.ds(sid * stripe, stripe)],
                    acc.at[pl.ds(sid * stripe, stripe)])
    pltpu.sync_copy(ones_in, onesv)
    plsc.subcore_barrier()

    def group(g, carry):
        pltpu.sync_copy(sidx.at[pl.ds(wid * CT + g * IG, IG)], sv)

        def chunk(j, carry2):
            pltpu.sync_copy(onesv, acc.at[sv.at[j]], add=True)
            return carry2
        lax.fori_loop(0, IG, chunk, 0)
        return carry
    lax.fori_loop(0, CT // IG, group, 0)
    plsc.subcore_barrier()
    pltpu.sync_copy(acc.at[pl.ds(sid * stripe, stripe)],
                    out_c.at[pl.ds(cid * N_PAD + sid * stripe, stripe)])


_cnt_call = functools.partial(
    pl.kernel, mesh=_mesh,
    out_type=[jax.ShapeDtypeStruct((2 * N_PAD, H), F32)],
    scratch_types=[
        pltpu.VMEM((IG, E_CHUNK), jnp.int32),
        pltpu.VMEM((E_CHUNK, H), F32),
        pltpu.VMEM_SHARED((N_PAD, H), F32),
    ],
)(_cnt_body)


def _cls_body(aidx, bidx, ta, tb, out, av, bv, ar, br, res, sema, semb):
    cid = lax.axis_index("c")
    sid = lax.axis_index("s")
    wid = cid * 16 + sid
    pltpu.sync_copy(aidx.at[pl.ds(wid * 32, 32)], av)
    pltpu.sync_copy(bidx.at[pl.ds(wid * 32, 32)], bv)

    def chunk(j, carry):
        ca = pltpu.async_copy(ta.at[av.at[j]], ar, sema)
        cb = pltpu.async_copy(tb.at[bv.at[j]], br, semb)
        ca.wait()
        cb.wait()

        def row(r, c2):
            acc = ar[r, pl.ds(0, 16)] * br[r, pl.ds(0, 16)]
            for k in range(1, 8):
                acc = acc + ar[r, pl.ds(k * 16, 16)] * br[r, pl.ds(k * 16, 16)]
            res[r] = acc
            return c2
        lax.fori_loop(0, E_CHUNK, row, 0)
        pltpu.sync_copy(res, out.at[pl.ds(wid * CTL * E_CHUNK + j * E_CHUNK,
                                          E_CHUNK)])
        return carry
    lax.fori_loop(0, CTL, chunk, 0)


_cls_call = functools.partial(
    pl.kernel, mesh=_mesh,
    out_type=[jax.ShapeDtypeStruct((EL_PAD, 16), F32)],
    scratch_types=[
        pltpu.VMEM((32, E_CHUNK), jnp.int32),
        pltpu.VMEM((32, E_CHUNK), jnp.int32),
        pltpu.VMEM((E_CHUNK, H), F32),
        pltpu.VMEM((E_CHUNK, H), F32),
        pltpu.VMEM((E_CHUNK, 16), F32),
        pltpu.SemaphoreType.DMA,
        pltpu.SemaphoreType.DMA,
    ],
)(_cls_body)


def _cls_reduce(p):
    def body(p_ref, o_ref):
        o_ref[...] = jnp.sum(p_ref[...], axis=1)
    return pl.pallas_call(
        body,
        grid=(100,),
        in_specs=[pl.BlockSpec((1024, 16), lambda i: (i, 0))],
        out_specs=pl.BlockSpec((1024,), lambda i: (i,)),
        out_shape=jax.ShapeDtypeStruct((EL_PAD,), F32),
    )(p)


def _encode(x, wt, b2, emb):
    def body(x_ref, w_ref, b_ref, e_ref, o_ref):
        o_ref[...] = (jnp.dot(x_ref[...], w_ref[...],
                              preferred_element_type=F32)
                      + b_ref[...] + e_ref[...])
    return pl.pallas_call(
        body,
        grid=(10,),
        in_specs=[pl.BlockSpec((1000, H), lambda i: (i, 0)),
                  pl.BlockSpec((H, H), lambda i: (0, 0)),
                  pl.BlockSpec((1, H), lambda i: (0, 0)),
                  pl.BlockSpec((1000, H), lambda i: (i, 0))],
        out_specs=pl.BlockSpec((1000, H), lambda i: (i, 0)),
        out_shape=jax.ShapeDtypeStruct((N_NODES, H), F32),
    )(x, wt, b2, emb)


def _combine(sums, counts, h_self, wlt, bl2, wrt, relu):
    def body(s_ref, c_ref, h_ref, wl_ref, b_ref, wr_ref, o_ref):
        c = c_ref[0, :, 0] + c_ref[1, :, 0]
        r = 1.0 / jnp.maximum(c, 1.0)
        s = s_ref[0] + s_ref[1]
        agg = s * r[:, None]
        y = (jnp.dot(agg, wl_ref[...], preferred_element_type=F32)
             + b_ref[...]
             + jnp.dot(h_ref[...], wr_ref[...], preferred_element_type=F32))
        o_ref[...] = jnp.maximum(y, 0.0) if relu else y
    return pl.pallas_call(
        body,
        grid=(10,),
        in_specs=[pl.BlockSpec((2, 1000, H), lambda i: (0, i, 0)),
                  pl.BlockSpec((2, 1000, H), lambda i: (0, i, 0)),
                  pl.BlockSpec((1000, H), lambda i: (i, 0)),
                  pl.BlockSpec((H, H), lambda i: (0, 0)),
                  pl.BlockSpec((1, H), lambda i: (0, 0)),
                  pl.BlockSpec((H, H), lambda i: (0, 0))],
        out_specs=pl.BlockSpec((1000, H), lambda i: (i, 0)),
        out_shape=jax.ShapeDtypeStruct((N_NODES, H), F32),
    )(sums, counts, h_self, wlt, bl2, wrt)


def kernel(x_user, x_movie, user_node_id, movie_node_id, edge_index,
           edge_label_index, W_user_lin, b_user_lin, W_movie_lin, b_movie_lin,
           user_emb, movie_emb, Wl1r, bl1r, Wr1r, Wl1v, bl1v, Wr1v,
           Wl2r, bl2r, Wr2r, Wl2v, bl2v, Wr2v):
    src = edge_index[0]
    dst = edge_index[1]
    padg = jnp.zeros((E_PAD - E,), jnp.int32)
    pads = jnp.full((E_PAD - E,), N_NODES, jnp.int32)
    g1 = jnp.concatenate([src, padg]).reshape(N_TILES * CT, E_CHUNK)
    s1 = jnp.concatenate([dst, pads]).reshape(N_TILES * CT, E_CHUNK)
    g2 = jnp.concatenate([dst, padg]).reshape(N_TILES * CT, E_CHUNK)
    s2 = jnp.concatenate([src, pads]).reshape(N_TILES * CT, E_CHUNK)
    padl = jnp.zeros((EL_PAD - EL,), jnp.int32)

    def _lidx(v):
        m = jnp.concatenate([v, padl]).reshape(N_TILES, CTL, E_CHUNK)
        return jnp.pad(m, ((0, 0), (0, 32 - CTL), (0, 0))).reshape(
            N_TILES * 32, E_CHUNK)
    la = _lidx(edge_label_index[0])
    lb = _lidx(edge_label_index[1])
    zacc = jnp.zeros((N_PAD, H), F32)
    ones128 = jnp.ones((E_CHUNK, H), F32)

    hu = _encode(x_user, W_user_lin.T, b_user_lin.reshape(1, H), user_emb)
    hm = _encode(x_movie, W_movie_lin.T, b_movie_lin.reshape(1, H), movie_emb)

    (sm,) = _seg_call(g1, s1, hu, zacc)
    (su,) = _seg_call(g2, s2, hm, zacc)
    (cm,) = _cnt_call(s1, zacc, ones128)
    (cu,) = _cnt_call(s2, zacc, ones128)
    sm = sm.reshape(2, N_PAD, H)[:, :N_NODES]
    su = su.reshape(2, N_PAD, H)[:, :N_NODES]
    cm = cm.reshape(2, N_PAD, H)[:, :N_NODES]
    cu = cu.reshape(2, N_PAD, H)[:, :N_NODES]

    hm1 = _combine(sm, cm, hm, Wl1r.T, bl1r.reshape(1, H), Wr1r.T, True)
    hu1 = _combine(su, cu, hu, Wl1v.T, bl1v.reshape(1, H), Wr1v.T, True)

    (sm2,) = _seg_call(g1, s1, hu1, zacc)
    (su2,) = _seg_call(g2, s2, hm1, zacc)
    sm2 = sm2.reshape(2, N_PAD, H)[:, :N_NODES]
    su2 = su2.reshape(2, N_PAD, H)[:, :N_NODES]

    hm2 = _combine(sm2, cm, hm1, Wl2r.T, bl2r.reshape(1, H), Wr2r.T, False)
    hu2 = _combine(su2, cu, hu1, Wl2v.T, bl2v.reshape(1, H), Wr2v.T, False)

    (partials,) = _cls_call(la, lb, hu2, hm2)
    out = _cls_reduce(partials)
    return out[:EL]


# speedup vs baseline: 2.5764x; 1.4405x over previous
import functools

import jax
import jax.numpy as jnp
from jax import lax
from jax.experimental import pallas as pl
from jax.experimental.pallas import tpu as pltpu
from jax.experimental.pallas import tpu_sc as plsc

F32 = jnp.float32
H = 128
N_NODES = 10000
N_PAD = 10240
E = 320000
E_CHUNK = 128
N_TILES = 32
CT = 80
IG = 16
E_PAD = N_TILES * CT * E_CHUNK
CNT_LEN = 10240
EL = 100000
CTL = 25
EL_PAD = N_TILES * CTL * E_CHUNK

_mesh = plsc.VectorSubcoreMesh(core_axis_name="c", subcore_axis_name="s")


CT2 = 160


def _seg_body(gidx, sidx, table, zacc, out_s,
              gv, sv, rows0, rows1, acc, sem0, sem1):
    cid = lax.axis_index("c")
    sid = lax.axis_index("s")
    base = cid * (16 * CT2) + sid * CT2
    stripe = N_PAD // 16
    pltpu.sync_copy(zacc.at[pl.ds(sid * stripe, stripe)],
                    acc.at[pl.ds(sid * stripe, stripe)])
    plsc.subcore_barrier()

    def group(g, carry):
        pltpu.sync_copy(gidx.at[pl.ds(base + g * IG, IG)], gv)
        pltpu.sync_copy(sidx.at[pl.ds(base + g * IG, IG)], sv)
        pltpu.async_copy(table.at[gv.at[0]], rows0, sem0)

        def pair(t, carry2):
            j0 = 2 * t
            pltpu.make_async_copy(table.at[gv.at[j0]], rows0, sem0).wait()
            ca = pltpu.async_copy(table.at[gv.at[j0 + 1]], rows1, sem1)
            pltpu.sync_copy(rows0, acc.at[sv.at[j0]], add=True)
            ca.wait()
            pltpu.async_copy(table.at[gv.at[j0 + 2]], rows0, sem0)
            pltpu.sync_copy(rows1, acc.at[sv.at[j0 + 1]], add=True)
            return carry2
        lax.fori_loop(0, IG // 2 - 1, pair, 0)
        pltpu.make_async_copy(table.at[gv.at[IG - 2]], rows0, sem0).wait()
        cb = pltpu.async_copy(table.at[gv.at[IG - 1]], rows1, sem1)
        pltpu.sync_copy(rows0, acc.at[sv.at[IG - 2]], add=True)
        cb.wait()
        pltpu.sync_copy(rows1, acc.at[sv.at[IG - 1]], add=True)
        return carry
    lax.fori_loop(0, CT2 // IG, group, 0)
    plsc.subcore_barrier()
    pltpu.sync_copy(acc.at[pl.ds(sid * stripe, stripe)],
                    out_s.at[pl.ds(cid * N_PAD + sid * stripe, stripe)])


_seg_call = functools.partial(
    pl.kernel, mesh=_mesh,
    out_type=[jax.ShapeDtypeStruct((2 * N_PAD, H), F32)],
    scratch_types=[
        pltpu.VMEM((IG, E_CHUNK), jnp.int32),
        pltpu.VMEM((IG, E_CHUNK), jnp.int32),
        pltpu.VMEM((E_CHUNK, H), F32),
        pltpu.VMEM((E_CHUNK, H), F32),
        pltpu.VMEM_SHARED((N_PAD, H), F32),
        pltpu.SemaphoreType.DMA,
        pltpu.SemaphoreType.DMA,
    ],
)(_seg_body)


def _cnt_body(sidx, zacc, ones_in, out_c, sv, onesv, acc, sem):
    cid = lax.axis_index("c")
    sid = lax.axis_index("s")
    base = cid * (16 * CT2) + sid * CT2
    stripe = N_PAD // 16
    pltpu.sync_copy(zacc.at[pl.ds(sid * stripe, stripe)],
                    acc.at[pl.ds(sid * stripe, stripe)])
    pltpu.sync_copy(ones_in, onesv)
    plsc.subcore_barrier()

    def group(g, carry):
        pltpu.sync_copy(sidx.at[pl.ds(base + g * IG, IG)], sv)

        def issue(j, carry2):
            pltpu.async_copy(onesv, acc.at[sv.at[j]], sem, add=True)
            return carry2
        lax.fori_loop(0, IG, issue, 0)

        def drain(j, carry2):
            pltpu.make_async_copy(onesv, acc.at[sv.at[j]], sem).wait()
            return carry2
        lax.fori_loop(0, IG, drain, 0)
        return carry
    lax.fori_loop(0, CT2 // IG, group, 0)
    plsc.subcore_barrier()
    pltpu.sync_copy(acc.at[pl.ds(sid * stripe, stripe)],
                    out_c.at[pl.ds(cid * N_PAD + sid * stripe, stripe)])


_cnt_call = functools.partial(
    pl.kernel, mesh=_mesh,
    out_type=[jax.ShapeDtypeStruct((2 * N_PAD, H), F32)],
    scratch_types=[
        pltpu.VMEM((IG, E_CHUNK), jnp.int32),
        pltpu.VMEM((E_CHUNK, H), F32),
        pltpu.VMEM_SHARED((N_PAD, H), F32),
        pltpu.SemaphoreType.DMA,
    ],
)(_cnt_body)


def _cls_body(aidx, bidx, ta, tb, out, av, bv, ar, br, res, sema, semb):
    cid = lax.axis_index("c")
    sid = lax.axis_index("s")
    wid = cid * 16 + sid
    pltpu.sync_copy(aidx.at[pl.ds(wid * 32, 32)], av)
    pltpu.sync_copy(bidx.at[pl.ds(wid * 32, 32)], bv)

    def chunk(j, carry):
        ca = pltpu.async_copy(ta.at[av.at[j]], ar, sema)
        cb = pltpu.async_copy(tb.at[bv.at[j]], br, semb)
        ca.wait()
        cb.wait()

        def row(r, c2):
            acc = ar[r, pl.ds(0, 16)] * br[r, pl.ds(0, 16)]
            for k in range(1, 8):
                acc = acc + ar[r, pl.ds(k * 16, 16)] * br[r, pl.ds(k * 16, 16)]
            res[r] = acc
            return c2
        lax.fori_loop(0, E_CHUNK, row, 0)
        pltpu.sync_copy(res, out.at[pl.ds(wid * CTL * E_CHUNK + j * E_CHUNK,
                                          E_CHUNK)])
        return carry
    lax.fori_loop(0, CTL, chunk, 0)


_cls_call = functools.partial(
    pl.kernel, mesh=_mesh,
    out_type=[jax.ShapeDtypeStruct((EL_PAD, 16), F32)],
    scratch_types=[
        pltpu.VMEM((32, E_CHUNK), jnp.int32),
        pltpu.VMEM((32, E_CHUNK), jnp.int32),
        pltpu.VMEM((E_CHUNK, H), F32),
        pltpu.VMEM((E_CHUNK, H), F32),
        pltpu.VMEM((E_CHUNK, 16), F32),
        pltpu.SemaphoreType.DMA,
        pltpu.SemaphoreType.DMA,
    ],
)(_cls_body)


def _cls_reduce(p):
    def body(p_ref, o_ref):
        o_ref[...] = jnp.sum(p_ref[...], axis=1)
    return pl.pallas_call(
        body,
        grid=(100,),
        in_specs=[pl.BlockSpec((1024, 16), lambda i: (i, 0))],
        out_specs=pl.BlockSpec((1024,), lambda i: (i,)),
        out_shape=jax.ShapeDtypeStruct((EL_PAD,), F32),
    )(p)


def _encode(x, wt, b2, emb):
    def body(x_ref, w_ref, b_ref, e_ref, o_ref):
        o_ref[...] = (jnp.dot(x_ref[...], w_ref[...],
                              preferred_element_type=F32)
                      + b_ref[...] + e_ref[...])
    return pl.pallas_call(
        body,
        grid=(10,),
        in_specs=[pl.BlockSpec((1000, H), lambda i: (i, 0)),
                  pl.BlockSpec((H, H), lambda i: (0, 0)),
                  pl.BlockSpec((1, H), lambda i: (0, 0)),
                  pl.BlockSpec((1000, H), lambda i: (i, 0))],
        out_specs=pl.BlockSpec((1000, H), lambda i: (i, 0)),
        out_shape=jax.ShapeDtypeStruct((N_NODES, H), F32),
    )(x, wt, b2, emb)


def _combine(sums, counts, h_self, wlt, bl2, wrt, relu):
    def body(s_ref, c_ref, h_ref, wl_ref, b_ref, wr_ref, o_ref):
        c = c_ref[:, 0]
        r = 1.0 / jnp.maximum(c, 1.0)
        agg = s_ref[...] * r[:, None]
        y = (jnp.dot(agg, wl_ref[...], preferred_element_type=F32)
             + b_ref[...]
             + jnp.dot(h_ref[...], wr_ref[...], preferred_element_type=F32))
        o_ref[...] = jnp.maximum(y, 0.0) if relu else y
    return pl.pallas_call(
        body,
        grid=(10,),
        in_specs=[pl.BlockSpec((1000, H), lambda i: (i, 0)),
                  pl.BlockSpec((1000, H), lambda i: (i, 0)),
                  pl.BlockSpec((1000, H), lambda i: (i, 0)),
                  pl.BlockSpec((H, H), lambda i: (0, 0)),
                  pl.BlockSpec((1, H), lambda i: (0, 0)),
                  pl.BlockSpec((H, H), lambda i: (0, 0))],
        out_specs=pl.BlockSpec((1000, H), lambda i: (i, 0)),
        out_shape=jax.ShapeDtypeStruct((N_NODES, H), F32),
    )(sums, counts, h_self, wlt, bl2, wrt)


def kernel(x_user, x_movie, user_node_id, movie_node_id, edge_index,
           edge_label_index, W_user_lin, b_user_lin, W_movie_lin, b_movie_lin,
           user_emb, movie_emb, Wl1r, bl1r, Wr1r, Wl1v, bl1v, Wr1v,
           Wl2r, bl2r, Wr2r, Wl2v, bl2v, Wr2v):
    src = edge_index[0]
    dst = edge_index[1]
    padg = jnp.zeros((E_PAD - E,), jnp.int32)
    pads = jnp.full((E_PAD - E,), N_NODES, jnp.int32)
    g1 = jnp.concatenate([src, padg])
    s1 = jnp.concatenate([dst, pads])
    g2 = jnp.concatenate([dst, padg]) + N_NODES
    s2 = jnp.concatenate([src, pads])
    g_all = jnp.concatenate([g1, g2]).reshape(2 * 16 * CT2, E_CHUNK)
    s_all = jnp.concatenate([s1, s2]).reshape(2 * 16 * CT2, E_CHUNK)
    padl = jnp.zeros((EL_PAD - EL,), jnp.int32)

    def _lidx(v):
        m = jnp.concatenate([v, padl]).reshape(N_TILES, CTL, E_CHUNK)
        return jnp.pad(m, ((0, 0), (0, 32 - CTL), (0, 0))).reshape(
            N_TILES * 32, E_CHUNK)
    la = _lidx(edge_label_index[0])
    lb = _lidx(edge_label_index[1])
    zacc = jnp.zeros((N_PAD, H), F32)
    ones128 = jnp.ones((E_CHUNK, H), F32)

    hu = _encode(x_user, W_user_lin.T, b_user_lin.reshape(1, H), user_emb)
    hm = _encode(x_movie, W_movie_lin.T, b_movie_lin.reshape(1, H), movie_emb)

    (s12,) = _seg_call(g_all, s_all, jnp.concatenate([hu, hm]), zacc)
    (c12,) = _cnt_call(s_all, zacc, ones128)
    sm = s12[:N_NODES]
    su = s12[N_PAD:N_PAD + N_NODES]
    cm = c12[:N_NODES]
    cu = c12[N_PAD:N_PAD + N_NODES]

    hm1 = _combine(sm, cm, hm, Wl1r.T, bl1r.reshape(1, H), Wr1r.T, True)
    hu1 = _combine(su, cu, hu, Wl1v.T, bl1v.reshape(1, H), Wr1v.T, True)

    (s34,) = _seg_call(g_all, s_all, jnp.concatenate([hu1, hm1]), zacc)
    sm2 = s34[:N_NODES]
    su2 = s34[N_PAD:N_PAD + N_NODES]

    hm2 = _combine(sm2, cm, hm1, Wl2r.T, bl2r.reshape(1, H), Wr2r.T, False)
    hu2 = _combine(su2, cu, hu1, Wl2v.T, bl2v.reshape(1, H), Wr2v.T, False)

    (partials,) = _cls_call(la, lb, hu2, hm2)
    out = _cls_reduce(partials)
    return out[:EL]


# speedup vs baseline: 2.6239x; 1.0184x over previous
import functools

import jax
import jax.numpy as jnp
from jax import lax
from jax.experimental import pallas as pl
from jax.experimental.pallas import tpu as pltpu
from jax.experimental.pallas import tpu_sc as plsc

F32 = jnp.float32
H = 128
N_NODES = 10000
N_PAD = 10240
E = 320000
E_CHUNK = 128
N_TILES = 32
CT = 80
IG = 16
E_PAD = N_TILES * CT * E_CHUNK
CNT_LEN = 10240
EL = 100000
CTL = 25
EL_PAD = N_TILES * CTL * E_CHUNK

_mesh = plsc.VectorSubcoreMesh(core_axis_name="c", subcore_axis_name="s")


CH = 64
SEG_CT = 320
SEG_IG = 8


def _seg_body(gidx, sidx, table, zacc, padidx, out_s,
              gv, sv, r0, r1, r2, r3, padv, acc,
              g0, g1, g2, g3, s0, s1, s2, s3):
    cid = lax.axis_index("c")
    sid = lax.axis_index("s")
    base = cid * (16 * SEG_CT) + sid * SEG_CT
    stripe = N_PAD // 16
    pltpu.sync_copy(zacc.at[pl.ds(sid * stripe, stripe)],
                    acc.at[pl.ds(sid * stripe, stripe)])
    pltpu.sync_copy(padidx, padv)
    plsc.subcore_barrier()
    bufs = ((r0, g0, s0), (r1, g1, s1), (r2, g2, s2), (r3, g3, s3))
    for (rb, _, sb) in bufs:
        pltpu.async_copy(rb, acc.at[padv], sb, add=True)

    def group(g, carry):
        pltpu.sync_copy(gidx.at[pl.ds(base + g * SEG_IG, SEG_IG)], gv)
        pltpu.sync_copy(sidx.at[pl.ds(base + g * SEG_IG, SEG_IG)], sv)
        for k in (0, 1):
            rb, gb, sb = bufs[k]
            pltpu.make_async_copy(rb, acc.at[padv], sb).wait()
            pltpu.async_copy(table.at[gv.at[k]], rb, gb)
        for pp in range(4):
            cur = bufs[:2] if pp % 2 == 0 else bufs[2:]
            nxt = bufs[2:] if pp % 2 == 0 else bufs[:2]
            if pp < 3:
                for k in (0, 1):
                    rb, gb, sb = nxt[k]
                    pltpu.make_async_copy(rb, acc.at[padv], sb).wait()
                    pltpu.async_copy(table.at[gv.at[2 * pp + 2 + k]], rb, gb)
            for k in (0, 1):
                rb, gb, sb = cur[k]
                pltpu.make_async_copy(table.at[gv.at[2 * pp + k]],
                                      rb, gb).wait()
                pltpu.async_copy(rb, acc.at[sv.at[2 * pp + k]], sb, add=True)
        return carry
    lax.fori_loop(0, SEG_CT // SEG_IG, group, 0)
    for (rb, _, sb) in bufs:
        pltpu.make_async_copy(rb, acc.at[padv], sb).wait()
    plsc.subcore_barrier()
    pltpu.sync_copy(acc.at[pl.ds(sid * stripe, stripe)],
                    out_s.at[pl.ds(cid * N_PAD + sid * stripe, stripe)])


_seg_call = functools.partial(
    pl.kernel, mesh=_mesh,
    out_type=[jax.ShapeDtypeStruct((2 * N_PAD, H), F32)],
    scratch_types=[
        pltpu.VMEM((SEG_IG, CH), jnp.int32),
        pltpu.VMEM((SEG_IG, CH), jnp.int32),
        pltpu.VMEM((CH, H), F32),
        pltpu.VMEM((CH, H), F32),
        pltpu.VMEM((CH, H), F32),
        pltpu.VMEM((CH, H), F32),
        pltpu.VMEM((CH,), jnp.int32),
        pltpu.VMEM_SHARED((N_PAD, H), F32),
        pltpu.SemaphoreType.DMA,
        pltpu.SemaphoreType.DMA,
        pltpu.SemaphoreType.DMA,
        pltpu.SemaphoreType.DMA,
        pltpu.SemaphoreType.DMA,
        pltpu.SemaphoreType.DMA,
        pltpu.SemaphoreType.DMA,
        pltpu.SemaphoreType.DMA,
    ],
)(_seg_body)


def _cnt_body(sidx, zacc, ones_in, out_c, sv, onesv, acc, sem):
    cid = lax.axis_index("c")
    sid = lax.axis_index("s")
    base = cid * (16 * SEG_CT) + sid * SEG_CT
    stripe = N_PAD // 16
    pltpu.sync_copy(zacc.at[pl.ds(sid * stripe, stripe)],
                    acc.at[pl.ds(sid * stripe, stripe)])
    pltpu.sync_copy(ones_in, onesv)
    plsc.subcore_barrier()

    def group(g, carry):
        pltpu.sync_copy(sidx.at[pl.ds(base + g * SEG_IG, SEG_IG)], sv)

        def issue(j, carry2):
            pltpu.async_copy(onesv, acc.at[sv.at[j]], sem, add=True)
            return carry2
        lax.fori_loop(0, SEG_IG, issue, 0)

        def drain(j, carry2):
            pltpu.make_async_copy(onesv, acc.at[sv.at[j]], sem).wait()
            return carry2
        lax.fori_loop(0, SEG_IG, drain, 0)
        return carry
    lax.fori_loop(0, SEG_CT // SEG_IG, group, 0)
    plsc.subcore_barrier()
    pltpu.sync_copy(acc.at[pl.ds(sid * stripe, stripe)],
                    out_c.at[pl.ds(cid * N_PAD + sid * stripe, stripe)])


_cnt_call = functools.partial(
    pl.kernel, mesh=_mesh,
    out_type=[jax.ShapeDtypeStruct((2 * N_PAD, H), F32)],
    scratch_types=[
        pltpu.VMEM((SEG_IG, CH), jnp.int32),
        pltpu.VMEM((CH, H), F32),
        pltpu.VMEM_SHARED((N_PAD, H), F32),
        pltpu.SemaphoreType.DMA,
    ],
)(_cnt_body)


def _cls_body(aidx, bidx, ta, tb, out, av, bv, ar, br, res, sema, semb):
    cid = lax.axis_index("c")
    sid = lax.axis_index("s")
    wid = cid * 16 + sid
    pltpu.sync_copy(aidx.at[pl.ds(wid * 32, 32)], av)
    pltpu.sync_copy(bidx.at[pl.ds(wid * 32, 32)], bv)

    def chunk(j, carry):
        ca = pltpu.async_copy(ta.at[av.at[j]], ar, sema)
        cb = pltpu.async_copy(tb.at[bv.at[j]], br, semb)
        ca.wait()
        cb.wait()

        def row(r, c2):
            acc = ar[r, pl.ds(0, 16)] * br[r, pl.ds(0, 16)]
            for k in range(1, 8):
                acc = acc + ar[r, pl.ds(k * 16, 16)] * br[r, pl.ds(k * 16, 16)]
            res[r] = acc
            return c2
        lax.fori_loop(0, E_CHUNK, row, 0)
        pltpu.sync_copy(res, out.at[pl.ds(wid * CTL * E_CHUNK + j * E_CHUNK,
                                          E_CHUNK)])
        return carry
    lax.fori_loop(0, CTL, chunk, 0)


_cls_call = functools.partial(
    pl.kernel, mesh=_mesh,
    out_type=[jax.ShapeDtypeStruct((EL_PAD, 16), F32)],
    scratch_types=[
        pltpu.VMEM((32, E_CHUNK), jnp.int32),
        pltpu.VMEM((32, E_CHUNK), jnp.int32),
        pltpu.VMEM((E_CHUNK, H), F32),
        pltpu.VMEM((E_CHUNK, H), F32),
        pltpu.VMEM((E_CHUNK, 16), F32),
        pltpu.SemaphoreType.DMA,
        pltpu.SemaphoreType.DMA,
    ],
)(_cls_body)


def _cls_reduce(p):
    def body(p_ref, o_ref):
        o_ref[...] = jnp.sum(p_ref[...], axis=1)
    return pl.pallas_call(
        body,
        grid=(100,),
        in_specs=[pl.BlockSpec((1024, 16), lambda i: (i, 0))],
        out_specs=pl.BlockSpec((1024,), lambda i: (i,)),
        out_shape=jax.ShapeDtypeStruct((EL_PAD,), F32),
    )(p)


def _encode(x, wt, b2, emb):
    def body(x_ref, w_ref, b_ref, e_ref, o_ref):
        o_ref[...] = (jnp.dot(x_ref[...], w_ref[...],
                              preferred_element_type=F32)
                      + b_ref[...] + e_ref[...])
    return pl.pallas_call(
        body,
        grid=(10,),
        in_specs=[pl.BlockSpec((1000, H), lambda i: (i, 0)),
                  pl.BlockSpec((H, H), lambda i: (0, 0)),
                  pl.BlockSpec((1, H), lambda i: (0, 0)),
                  pl.BlockSpec((1000, H), lambda i: (i, 0))],
        out_specs=pl.BlockSpec((1000, H), lambda i: (i, 0)),
        out_shape=jax.ShapeDtypeStruct((N_NODES, H), F32),
    )(x, wt, b2, emb)


def _combine(sums, counts, h_self, wlt, bl2, wrt, relu):
    def body(s_ref, c_ref, h_ref, wl_ref, b_ref, wr_ref, o_ref):
        c = c_ref[:, 0]
        r = 1.0 / jnp.maximum(c, 1.0)
        agg = s_ref[...] * r[:, None]
        y = (jnp.dot(agg, wl_ref[...], preferred_element_type=F32)
             + b_ref[...]
             + jnp.dot(h_ref[...], wr_ref[...], preferred_element_type=F32))
        o_ref[...] = jnp.maximum(y, 0.0) if relu else y
    return pl.pallas_call(
        body,
        grid=(10,),
        in_specs=[pl.BlockSpec((1000, H), lambda i: (i, 0)),
                  pl.BlockSpec((1000, H), lambda i: (i, 0)),
                  pl.BlockSpec((1000, H), lambda i: (i, 0)),
                  pl.BlockSpec((H, H), lambda i: (0, 0)),
                  pl.BlockSpec((1, H), lambda i: (0, 0)),
                  pl.BlockSpec((H, H), lambda i: (0, 0))],
        out_specs=pl.BlockSpec((1000, H), lambda i: (i, 0)),
        out_shape=jax.ShapeDtypeStruct((N_NODES, H), F32),
    )(sums, counts, h_self, wlt, bl2, wrt)


def kernel(x_user, x_movie, user_node_id, movie_node_id, edge_index,
           edge_label_index, W_user_lin, b_user_lin, W_movie_lin, b_movie_lin,
           user_emb, movie_emb, Wl1r, bl1r, Wr1r, Wl1v, bl1v, Wr1v,
           Wl2r, bl2r, Wr2r, Wl2v, bl2v, Wr2v):
    src = edge_index[0]
    dst = edge_index[1]
    padg = jnp.zeros((E_PAD - E,), jnp.int32)
    pads = jnp.full((E_PAD - E,), N_NODES, jnp.int32)
    g1 = jnp.concatenate([src, padg])
    s1 = jnp.concatenate([dst, pads])
    g2 = jnp.concatenate([dst, padg]) + N_NODES
    s2 = jnp.concatenate([src, pads])
    g_all = jnp.concatenate([g1, g2]).reshape(2 * 16 * SEG_CT, CH)
    s_all = jnp.concatenate([s1, s2]).reshape(2 * 16 * SEG_CT, CH)
    padl = jnp.zeros((EL_PAD - EL,), jnp.int32)

    def _lidx(v):
        m = jnp.concatenate([v, padl]).reshape(N_TILES, CTL, E_CHUNK)
        return jnp.pad(m, ((0, 0), (0, 32 - CTL), (0, 0))).reshape(
            N_TILES * 32, E_CHUNK)
    la = _lidx(edge_label_index[0])
    lb = _lidx(edge_label_index[1])
    zacc = jnp.zeros((N_PAD, H), F32)
    ones_rows = jnp.ones((CH, H), F32)
    padidx = jnp.full((CH,), N_NODES, jnp.int32)

    hu = _encode(x_user, W_user_lin.T, b_user_lin.reshape(1, H), user_emb)
    hm = _encode(x_movie, W_movie_lin.T, b_movie_lin.reshape(1, H), movie_emb)

    (s12,) = _seg_call(g_all, s_all, jnp.concatenate([hu, hm]), zacc, padidx)
    (c12,) = _cnt_call(s_all, zacc, ones_rows)
    sm = s12[:N_NODES]
    su = s12[N_PAD:N_PAD + N_NODES]
    cm = c12[:N_NODES]
    cu = c12[N_PAD:N_PAD + N_NODES]

    hm1 = _combine(sm, cm, hm, Wl1r.T, bl1r.reshape(1, H), Wr1r.T, True)
    hu1 = _combine(su, cu, hu, Wl1v.T, bl1v.reshape(1, H), Wr1v.T, True)

    (s34,) = _seg_call(g_all, s_all, jnp.concatenate([hu1, hm1]), zacc,
                       padidx)
    sm2 = s34[:N_NODES]
    su2 = s34[N_PAD:N_PAD + N_NODES]

    hm2 = _combine(sm2, cm, hm1, Wl2r.T, bl2r.reshape(1, H), Wr2r.T, False)
    hu2 = _combine(su2, cu, hu1, Wl2v.T, bl2v.reshape(1, H), Wr2v.T, False)

    (partials,) = _cls_call(la, lb, hu2, hm2)
    out = _cls_reduce(partials)
    return out[:EL]
